# baseline (speedup 1.0000x reference)
import functools

import jax
import jax.numpy as jnp
from jax import lax
from jax.experimental import pallas as pl
from jax.experimental.pallas import tpu as pltpu
from jax.experimental.pallas import tpu_sc as plsc

V = 1000
VP = 1024
D = 32
T = 8
B = 4096

NC, NS = 2, 16
NW = NC * NS
BTOT = B * T
BPW = BTOT // NW
ROWS = 16
NBUF = 4
NCH = BPW // ROWS
NGRP = NCH // NBUF
LANES = 16


def _table_body(emb_ref, w_ref, pos_ref, bias_ref, out_ref, acc_ref):
    t = pl.program_id(0)

    @pl.when(t == 0)
    def _():
        acc_ref[:] = lax.dot_general(
            emb_ref[:], w_ref[:], (((1,), (0,)), ((), ())),
            precision=lax.Precision.HIGHEST,
            preferred_element_type=jnp.float32,
        )

    posr = lax.dot_general(
        pos_ref[pl.ds(t, 1), :], w_ref[:], (((1,), (0,)), ((), ())),
        precision=lax.Precision.HIGHEST,
        preferred_element_type=jnp.float32,
    )
    out_ref[:] = acc_ref[:] + posr + bias_ref[:]


def _build_table(embedding, lm_head_w, positional_embedding, lm_head_b):
    return pl.pallas_call(
        _table_body,
        grid=(T,),
        in_specs=[
            pl.BlockSpec((V, D), lambda t: (0, 0)),
            pl.BlockSpec((D, V), lambda t: (0, 0)),
            pl.BlockSpec((T, D), lambda t: (0, 0)),
            pl.BlockSpec((1, V), lambda t: (0, 0)),
        ],
        out_specs=pl.BlockSpec((V, V), lambda t: (t, 0)),
        out_shape=jax.ShapeDtypeStruct((T * V, V), jnp.float32),
        scratch_shapes=[pltpu.VMEM((V, V), jnp.float32)],
    )(embedding, lm_head_w, positional_embedding, lm_head_b.reshape(1, V))


def _sc_gather_body(table_hbm, idx_hbm, out_hbm, idx_v, bufs_v, gsem, ssem):
    wid = lax.axis_index("s") * NC + lax.axis_index("c")
    base = wid * BPW

    pltpu.sync_copy(idx_hbm.at[pl.ds(base, BPW)], idx_v)

    toff = (lax.iota(jnp.int32, LANES) % T) * V

    def _addt(i, carry):
        idx_v[pl.ds(i * LANES, LANES)] = idx_v[pl.ds(i * LANES, LANES)] + toff
        return carry

    lax.fori_loop(0, BPW // LANES, _addt, 0)

    def issue_gather(chunk, b):
        pltpu.async_copy(
            table_hbm.at[idx_v.at[pl.ds(chunk * ROWS, ROWS)]], bufs_v.at[b], gsem
        )

    def wait_gather(chunk, b):
        pltpu.make_async_copy(
            table_hbm.at[idx_v.at[pl.ds(chunk * ROWS, ROWS)]], bufs_v.at[b], gsem
        ).wait()

    def issue_scatter(chunk, b):
        pltpu.async_copy(
            bufs_v.at[b], out_hbm.at[pl.ds(base + chunk * ROWS, ROWS)], ssem
        )

    def wait_scatter(chunk, b):
        pltpu.make_async_copy(
            bufs_v.at[b], out_hbm.at[pl.ds(base + chunk * ROWS, ROWS)], ssem
        ).wait()

    for b in range(NBUF):
        issue_gather(b, b)

    def _group(j, carry):
        g0 = j * NBUF
        for b in range(NBUF):
            wait_gather(g0 + b, b)
        for b in range(NBUF):
            issue_scatter(g0 + b, b)
        for b in range(NBUF):
            wait_scatter(g0 + b, b)
        for b in range(NBUF):
            issue_gather(g0 + NBUF + b, b)
        return carry

    lax.fori_loop(0, NGRP - 1, _group, 0)

    g0 = (NGRP - 1) * NBUF
    for b in range(NBUF):
        wait_gather(g0 + b, b)
    for b in range(NBUF):
        issue_scatter(g0 + b, b)
    for b in range(NBUF):
        wait_scatter(g0 + b, b)


@functools.cache
def _sc_gather():
    mesh = plsc.VectorSubcoreMesh(
        core_axis_name="c", subcore_axis_name="s", num_cores=NC, num_subcores=NS
    )
    return pl.kernel(
        _sc_gather_body,
        out_type=jax.ShapeDtypeStruct((BTOT, V), jnp.float32),
        mesh=mesh,
        scratch_types=[
            pltpu.VMEM((BPW,), jnp.int32),
            pltpu.VMEM((NBUF, ROWS, V), jnp.float32),
            pltpu.SemaphoreType.DMA,
            pltpu.SemaphoreType.DMA,
        ],
        compiler_params=pltpu.CompilerParams(use_tc_tiling_on_sc=False),
    )


def kernel(idx, embedding, positional_embedding, lm_head_w, lm_head_b):
    table8 = _build_table(embedding, lm_head_w, positional_embedding, lm_head_b)
    idx_flat = idx.reshape(BTOT).astype(jnp.int32)
    out = _sc_gather()(table8, idx_flat)
    return out.reshape(B, T, V)


# speedup vs baseline: 1.6463x; 1.6463x over previous
import functools

import jax
import jax.numpy as jnp
from jax import lax
from jax.experimental import pallas as pl
from jax.experimental.pallas import tpu as pltpu
from jax.experimental.pallas import tpu_sc as plsc

V = 1000
D = 32
T = 8
B = 4096

NC, NS = 2, 16
NW = NC * NS
BTOT = B * T
BPW = BTOT // NW
GROWS = 128
NG = BPW // GROWS
LANES = 16

BM = 2048
NBLK = BTOT // BM


def _emb8_body(emb_ref, pos_ref, out_ref):
    t = pl.program_id(0)
    out_ref[:] = emb_ref[:] + pos_ref[pl.ds(t, 1), :]


def _build_emb8(embedding, positional_embedding):
    return pl.pallas_call(
        _emb8_body,
        grid=(T,),
        in_specs=[
            pl.BlockSpec((V, D), lambda t: (0, 0)),
            pl.BlockSpec((T, D), lambda t: (0, 0)),
        ],
        out_specs=pl.BlockSpec((V, D), lambda t: (t, 0)),
        out_shape=jax.ShapeDtypeStruct((T * V, D), jnp.float32),
    )(embedding, positional_embedding)


def _sc_gather_body(emb8_hbm, idx_hbm, out_hbm, idx_v, rows_v, gsem, ssem):
    wid = lax.axis_index("s") * NC + lax.axis_index("c")
    base = wid * BPW

    pltpu.sync_copy(idx_hbm.at[pl.ds(base, BPW)], idx_v)

    toff = (lax.iota(jnp.int32, LANES) % T) * V

    def _addt(i, carry):
        idx_v[pl.ds(i * LANES, LANES)] = idx_v[pl.ds(i * LANES, LANES)] + toff
        return carry

    lax.fori_loop(0, BPW // LANES, _addt, 0)

    for g in range(NG):
        pltpu.async_copy(
            emb8_hbm.at[idx_v.at[pl.ds(g * GROWS, GROWS)]],
            rows_v.at[pl.ds(g * GROWS, GROWS)],
            gsem,
        )
    for g in range(NG):
        pltpu.make_async_copy(
            emb8_hbm.at[idx_v.at[pl.ds(g * GROWS, GROWS)]],
            rows_v.at[pl.ds(g * GROWS, GROWS)],
            gsem,
        ).wait()

    pltpu.async_copy(rows_v, out_hbm.at[pl.ds(base, BPW)], ssem)
    pltpu.make_async_copy(rows_v, out_hbm.at[pl.ds(base, BPW)], ssem).wait()


@functools.cache
def _sc_gather():
    mesh = plsc.VectorSubcoreMesh(
        core_axis_name="c", subcore_axis_name="s", num_cores=NC, num_subcores=NS
    )
    return pl.kernel(
        _sc_gather_body,
        out_type=jax.ShapeDtypeStruct((BTOT, D), jnp.float32),
        mesh=mesh,
        scratch_types=[
            pltpu.VMEM((BPW,), jnp.int32),
            pltpu.VMEM((BPW, D), jnp.float32),
            pltpu.SemaphoreType.DMA,
            pltpu.SemaphoreType.DMA,
        ],
        compiler_params=pltpu.CompilerParams(use_tc_tiling_on_sc=False),
    )


def _head_body(x_ref, w_ref, bias_ref, out_ref):
    out_ref[:] = lax.dot_general(
        x_ref[:], w_ref[:], (((1,), (0,)), ((), ())),
        precision=lax.Precision.HIGHEST,
        preferred_element_type=jnp.float32,
    ) + bias_ref[:]


def _head(x, lm_head_w, lm_head_b):
    return pl.pallas_call(
        _head_body,
        grid=(NBLK,),
        in_specs=[
            pl.BlockSpec((BM, D), lambda i: (i, 0)),
            pl.BlockSpec((D, V), lambda i: (0, 0)),
            pl.BlockSpec((1, V), lambda i: (0, 0)),
        ],
        out_specs=pl.BlockSpec((BM, V), lambda i: (i, 0)),
        out_shape=jax.ShapeDtypeStruct((BTOT, V), jnp.float32),
    )(x, lm_head_w, lm_head_b.reshape(1, V))


def kernel(idx, embedding, positional_embedding, lm_head_w, lm_head_b):
    emb8 = _build_emb8(embedding, positional_embedding)
    idx_flat = idx.reshape(BTOT).astype(jnp.int32)
    x = _sc_gather()(emb8, idx_flat)
    out = _head(x, lm_head_w, lm_head_b)
    return out.reshape(B, T, V)
